# baseline (speedup 1.0000x reference)
import dataclasses
import functools

import jax
import jax.numpy as jnp
from jax import lax
from jax.experimental import pallas as pl
from jax.experimental.pallas import tpu as pltpu
from jax.experimental.pallas import tpu_sc as plsc

_NC = 2
_NS = 16
_L = 16
_CHUNK = 128


def _make_sc_call(B, D, n_workers, bpw, nchunk):
    mesh = plsc.VectorSubcoreMesh(
        core_axis_name="c", subcore_axis_name="s",
        num_cores=_NC, num_subcores=_NS)

    cp = pltpu.CompilerParams()
    if "needs_layout_passes" in pltpu.CompilerParams.__dataclass_fields__:
        cp = dataclasses.replace(cp, needs_layout_passes=False)
    if "use_tc_tiling_on_sc" in pltpu.CompilerParams.__dataclass_fields__:
        cp = dataclasses.replace(cp, use_tc_tiling_on_sc=False)

    @functools.partial(
        pl.kernel,
        compiler_params=cp,
        out_type=jax.ShapeDtypeStruct((n_workers, bpw), jnp.float32),
        mesh=mesh,
        scratch_types=[
            pltpu.VMEM((nchunk, _CHUNK), jnp.int32),
            pltpu.VMEM((nchunk, _CHUNK), jnp.int32),
            pltpu.VMEM((bpw, D), jnp.float32),
            pltpu.VMEM((bpw, D), jnp.float32),
            pltpu.VMEM((bpw,), jnp.float32),
            pltpu.VMEM((bpw,), jnp.float32),
            pltpu.SemaphoreType.DMA,
            pltpu.SemaphoreType.DMA,
        ],
    )
    def run(i_hbm, j_hbm, de_hbm, u_hbm, v_hbm, o_hbm,
            idx_u, idx_v, urows, vrows, de_v, out_v, sem_u, sem_v):
        w = lax.axis_index("s") * _NC + lax.axis_index("c")
        pltpu.sync_copy(i_hbm.at[w], idx_u)
        pltpu.sync_copy(j_hbm.at[w], idx_v)
        pltpu.sync_copy(de_hbm.at[w], de_v)

        copies = []
        for c in range(nchunk):
            copies.append(pltpu.async_copy(
                u_hbm.at[idx_u.at[c]],
                urows.at[pl.ds(c * _CHUNK, _CHUNK)], sem_u))
            copies.append(pltpu.async_copy(
                v_hbm.at[idx_v.at[c]],
                vrows.at[pl.ds(c * _CHUNK, _CHUNK)], sem_v))
        for cp in copies:
            cp.wait()

        @pl.loop(0, bpw, step=_L)
        def _(r0):
            rows = r0 + lax.iota(jnp.int32, _L)
            acc0 = jnp.zeros((_L,), jnp.float32)
            acc1 = jnp.zeros((_L,), jnp.float32)
            for d in range(0, D, 2):
                c0 = jnp.full((_L,), d, jnp.int32)
                c1 = jnp.full((_L,), d + 1, jnp.int32)
                acc0 += (plsc.load_gather(urows, [rows, c0])
                         * plsc.load_gather(vrows, [rows, c0]))
                acc1 += (plsc.load_gather(urows, [rows, c1])
                         * plsc.load_gather(vrows, [rows, c1]))
            t = de_v[pl.ds(r0, _L)] * (acc0 + acc1)
            out_v[pl.ds(r0, _L)] = -1.0 / (1.0 + jnp.exp(-t))

        pltpu.sync_copy(out_v, o_hbm.at[w])

    return run


def kernel(pair, U, V):
    B = pair.shape[0]
    D = U.shape[1]
    n_workers = _NC * _NS
    bpw = B // n_workers
    nchunk = bpw // _CHUNK

    i = pair[:, 0].reshape(n_workers, nchunk, _CHUNK)
    j = pair[:, 1].reshape(n_workers, nchunk, _CHUNK)
    de = pair[:, 2].astype(jnp.float32).reshape(n_workers, bpw)

    run = _make_sc_call(B, D, n_workers, bpw, nchunk)
    out = run(i, j, de, U, V)
    return out.reshape(B, 1)
